# baseline (speedup 1.0000x reference)
import functools

import jax
import jax.numpy as jnp
import numpy as np
from jax import lax
from jax.experimental import pallas as pl
from jax.experimental.pallas import tpu as pltpu
from jax.experimental.pallas import tpu_sc as plsc

NUM_LEVELS = 16
BASE_RES = 16
MAX_RES = 2048
F = 2
HASHMAP = 2 ** 19
_b = np.exp((np.log(MAX_RES) - np.log(BASE_RES)) / (NUM_LEVELS - 1))
RESOLUTIONS = [int(np.floor(BASE_RES * _b ** i)) for i in range(NUM_LEVELS)]
TABLE_SIZES = [min(r * r, HASHMAP) for r in RESOLUTIONS]
P1 = 2654435761
P2 = 805459861
B = 1048576

P1_HI, P1_LO = P1 >> 16, P1 & 0xFFFF
P2_HI, P2_LO = P2 >> 16, P2 & 0xFFFF

NC, NS, L = 2, 16, 16
NW = NC * NS
PPW = B // NW
CH = 128
NCHUNK = PPW // CH
NVR = CH // L

N_RES = 7
N_STR = NUM_LEVELS - N_RES

_RBASE = []
_off = 0
for _l in range(N_RES):
    _RBASE.append(_off)
    _off += (TABLE_SIZES[_l] + 7) // 8 * 8
RES_ROWS = _off

R32 = [(1 << 32) % t for t in TABLE_SIZES]
R24 = [(1 << 24) % t for t in TABLE_SIZES]
R16 = [(1 << 16) % t for t in TABLE_SIZES]


def _hprod2(c, phi, plo, need_hi):
    a1 = c * jnp.int32(plo)
    a2 = c * jnp.int32(phi)
    lo = (a2 << 16) + a1
    hi = ((a2 + (a1 >> 16)) >> 16) if need_hi else None
    return lo, hi


def _corner_idx(lev, xlo, xhi, ylo, yhi):
    h_lo = xlo ^ ylo
    ts = TABLE_SIZES[lev]
    if ts == HASHMAP:
        return h_lo & jnp.int32(HASHMAP - 1)
    h_hi = xhi ^ yhi
    m = (h_hi * jnp.int32(R32[lev])
         + ((h_lo >> 24) & jnp.int32(0xFF)) * jnp.int32(R24[lev])
         + ((h_lo >> 16) & jnp.int32(0xFF)) * jnp.int32(R16[lev])
         + (h_lo & jnp.int32(0xFFFF)))
    return lax.rem(m, jnp.int32(ts))


def _level_corners(lev, xn, yn):
    res = RESOLUTIONS[lev]
    need_hi = TABLE_SIZES[lev] != HASHMAP
    xs = xn * jnp.float32(res)
    ys = yn * jnp.float32(res)
    xi = xs.astype(jnp.int32)
    yi = ys.astype(jnp.int32)
    wx = xs - xi.astype(jnp.float32)
    wy = ys - yi.astype(jnp.float32)
    x0 = jnp.minimum(xi, jnp.int32(res - 1))
    x1 = jnp.minimum(xi + jnp.int32(1), jnp.int32(res - 1))
    y0 = jnp.minimum(yi, jnp.int32(res - 1))
    y1 = jnp.minimum(yi + jnp.int32(1), jnp.int32(res - 1))
    x0lo, x0hi = _hprod2(x0, P1_HI, P1_LO, need_hi)
    x1lo, x1hi = _hprod2(x1, P1_HI, P1_LO, need_hi)
    y0lo, y0hi = _hprod2(y0, P2_HI, P2_LO, need_hi)
    y1lo, y1hi = _hprod2(y1, P2_HI, P2_LO, need_hi)
    c00 = _corner_idx(lev, x0lo, x0hi, y0lo, y0hi)
    c10 = _corner_idx(lev, x1lo, x1hi, y0lo, y0hi)
    c01 = _corner_idx(lev, x0lo, x0hi, y1lo, y1hi)
    c11 = _corner_idx(lev, x1lo, x1hi, y1lo, y1hi)
    return (c00, c10, c01, c11), wx, wy


def _bilerp(t00, t10, t01, t11, wx, wy):
    a = t00 + wx * (t10 - t00)
    b = t01 + wx * (t11 - t01)
    return a + wy * (b - a)


def _load_xy(x_buf, pids):
    px = plsc.load_gather(x_buf, [pids * jnp.int32(2)])
    py = plsc.load_gather(x_buf, [pids * jnp.int32(2) + jnp.int32(1)])
    xn = jnp.clip((px + jnp.float32(1.0)) * jnp.float32(0.5),
                  jnp.float32(0.0), jnp.float32(1.0))
    yn = jnp.clip((py + jnp.float32(1.0)) * jnp.float32(0.5),
                  jnp.float32(0.0), jnp.float32(1.0))
    return xn, yn


@functools.cache
def _build():
    mesh = plsc.VectorSubcoreMesh(core_axis_name="c", subcore_axis_name="s",
                                  num_cores=NC, num_subcores=NS)

    scratch = [pltpu.VMEM((RES_ROWS * F,), jnp.float32)]
    scratch += [pltpu.VMEM((CH * 2,), jnp.float32)]
    scratch += [pltpu.VMEM((CH, 2 * NUM_LEVELS), jnp.float32)]
    scratch += [pltpu.VMEM((CH,), jnp.int32) for _ in range(4 * N_STR)]
    scratch += [pltpu.VMEM((CH, 8), jnp.float32) for _ in range(4 * N_STR)]
    scratch += [pltpu.VMEM((2 * N_STR * CH,), jnp.float32)]
    scratch += [pltpu.SemaphoreType.DMA]

    @functools.partial(
        pl.kernel,
        out_type=jax.ShapeDtypeStruct((B, 2 * NUM_LEVELS), jnp.float32),
        mesh=mesh,
        scratch_types=scratch,
        compiler_params=pltpu.CompilerParams(
            needs_layout_passes=False, use_tc_tiling_on_sc=False),
    )
    def hash_embed(x_hbm, *rest):
        tabs = rest[:NUM_LEVELS]
        out_hbm = rest[NUM_LEVELS]
        sc = rest[NUM_LEVELS + 1:]
        res_tab = sc[0]
        x_buf = sc[1]
        out_buf = sc[2]
        idx_refs = sc[3:3 + 4 * N_STR]
        row_refs = sc[3 + 4 * N_STR:3 + 8 * N_STR]
        w_ref = sc[3 + 8 * N_STR]
        sem = sc[3 + 8 * N_STR + 1]

        wid = lax.axis_index("s") * NC + lax.axis_index("c")
        wbase = wid * jnp.int32(PPW)

        iota = lax.iota(jnp.int32, L)

        for l in range(N_RES):
            pltpu.sync_copy(
                tabs[l], res_tab.at[pl.ds(F * _RBASE[l], F * TABLE_SIZES[l])])

        def chunk_body(ci, carry):
            base = wbase + ci * jnp.int32(CH)
            pltpu.sync_copy(x_hbm.at[pl.ds(base * jnp.int32(2), CH * 2)], x_buf)

            def phase_a(v, c):
                pids = iota + v * jnp.int32(L)
                xn, yn = _load_xy(x_buf, pids)
                sl = pl.ds(v * jnp.int32(L), L)
                for l in range(N_RES, NUM_LEVELS):
                    k = (l - N_RES) * 4
                    corners, wx, wy = _level_corners(l, xn, yn)
                    for c4 in range(4):
                        idx_refs[k + c4][sl] = corners[c4]
                    w_ref[pl.ds(v * jnp.int32(L) + (l - N_RES) * CH, L)] = wx
                    w_ref[pl.ds(v * jnp.int32(L) + (N_STR + l - N_RES) * CH, L)] = wy
                return c

            lax.fori_loop(np.int32(0), np.int32(NVR), phase_a, np.int32(0))

            handles = []
            for l in range(N_RES, NUM_LEVELS):
                k = (l - N_RES) * 4
                for j in range(4):
                    handles.append(
                        pltpu.async_copy(tabs[l].at[idx_refs[k + j]],
                                         row_refs[k + j], sem))

            def phase_b(v, c):
                pids = iota + v * jnp.int32(L)
                xn, yn = _load_xy(x_buf, pids)
                for l in range(N_RES):
                    corners, wx, wy = _level_corners(l, xn, yn)
                    feats = []
                    for f in range(F):
                        off = jnp.int32(F * _RBASE[l] + f)
                        t = [plsc.load_gather(res_tab, [cc * jnp.int32(F) + off])
                             for cc in corners]
                        feats.append(_bilerp(t[0], t[1], t[2], t[3], wx, wy))
                    for f in range(F):
                        plsc.store_scatter(
                            out_buf, [pids, jnp.full((L,), 2 * l + f, jnp.int32)],
                            feats[f])
                return c

            lax.fori_loop(np.int32(0), np.int32(NVR), phase_b, np.int32(0))

            for h in handles:
                h.wait()

            def phase_c(v, c):
                pids = iota + v * jnp.int32(L)
                sl = pl.ds(v * jnp.int32(L), L)
                for l in range(N_RES, NUM_LEVELS):
                    k = (l - N_RES) * 4
                    wx = w_ref[pl.ds(v * jnp.int32(L) + (l - N_RES) * CH, L)]
                    wy = w_ref[pl.ds(v * jnp.int32(L) + (N_STR + l - N_RES) * CH, L)]
                    for f in range(F):
                        cf = jnp.full((L,), f, jnp.int32)
                        t = [plsc.load_gather(row_refs[k + c4], [pids, cf])
                             for c4 in range(4)]
                        val = _bilerp(t[0], t[1], t[2], t[3], wx, wy)
                        plsc.store_scatter(
                            out_buf, [pids, jnp.full((L,), 2 * l + f, jnp.int32)],
                            val)
                return c

            lax.fori_loop(np.int32(0), np.int32(NVR), phase_c, np.int32(0))

            pltpu.sync_copy(out_buf, out_hbm.at[pl.ds(base, CH)])
            return carry

        lax.fori_loop(np.int32(0), np.int32(NCHUNK), chunk_body, np.int32(0))

    return hash_embed


def kernel(x, table_0, table_1, table_2, table_3, table_4, table_5, table_6,
           table_7, table_8, table_9, table_10, table_11, table_12,
           table_13, table_14, table_15):
    tabs = [table_0, table_1, table_2, table_3, table_4, table_5, table_6,
            table_7, table_8, table_9, table_10, table_11, table_12,
            table_13, table_14, table_15]
    with jax.enable_x64(False):
        x_flat = x.reshape(B * 2)
        args = [t.reshape(-1) for t in tabs[:N_RES]]
        args += [jnp.pad(t, ((0, 0), (0, 8 - F))) for t in tabs[N_RES:]]
        return _build()(x_flat, *args)


# speedup vs baseline: 1.0743x; 1.0743x over previous
import functools

import jax
import jax.numpy as jnp
import numpy as np
from jax import lax
from jax.experimental import pallas as pl
from jax.experimental.pallas import tpu as pltpu
from jax.experimental.pallas import tpu_sc as plsc

NUM_LEVELS = 16
BASE_RES = 16
MAX_RES = 2048
F = 2
HASHMAP = 2 ** 19
_b = np.exp((np.log(MAX_RES) - np.log(BASE_RES)) / (NUM_LEVELS - 1))
RESOLUTIONS = [int(np.floor(BASE_RES * _b ** i)) for i in range(NUM_LEVELS)]
TABLE_SIZES = [min(r * r, HASHMAP) for r in RESOLUTIONS]
P1 = 2654435761
P2 = 805459861
B = 1048576

P1_HI, P1_LO = P1 >> 16, P1 & 0xFFFF
P2_HI, P2_LO = P2 >> 16, P2 & 0xFFFF

NC, NS, L = 2, 16, 16
NW = NC * NS
PPW = B // NW
CH = 128
NCHUNK = PPW // CH
NVR = CH // L

N_RES = 7
N_STR = NUM_LEVELS - N_RES

_RBASE = []
_off = 0
for _l in range(N_RES):
    _RBASE.append(_off)
    _off += (TABLE_SIZES[_l] + 7) // 8 * 8
RES_ROWS = _off

R32 = [(1 << 32) % t for t in TABLE_SIZES]
R24 = [(1 << 24) % t for t in TABLE_SIZES]
R16 = [(1 << 16) % t for t in TABLE_SIZES]


def _hprod2(c, phi, plo, need_hi):
    a1 = c * jnp.int32(plo)
    a2 = c * jnp.int32(phi)
    lo = (a2 << 16) + a1
    hi = ((a2 + (a1 >> 16)) >> 16) if need_hi else None
    return lo, hi


def _corner_idx(lev, xlo, xhi, ylo, yhi):
    h_lo = xlo ^ ylo
    ts = TABLE_SIZES[lev]
    if ts == HASHMAP:
        return h_lo & jnp.int32(HASHMAP - 1)
    h_hi = xhi ^ yhi
    m = (h_hi * jnp.int32(R32[lev])
         + ((h_lo >> 24) & jnp.int32(0xFF)) * jnp.int32(R24[lev])
         + ((h_lo >> 16) & jnp.int32(0xFF)) * jnp.int32(R16[lev])
         + (h_lo & jnp.int32(0xFFFF)))
    return lax.rem(m, jnp.int32(ts))


def _level_corners(lev, xn, yn):
    res = RESOLUTIONS[lev]
    need_hi = TABLE_SIZES[lev] != HASHMAP
    xs = xn * jnp.float32(res)
    ys = yn * jnp.float32(res)
    xi = xs.astype(jnp.int32)
    yi = ys.astype(jnp.int32)
    wx = xs - xi.astype(jnp.float32)
    wy = ys - yi.astype(jnp.float32)
    x0 = jnp.minimum(xi, jnp.int32(res - 1))
    x1 = jnp.minimum(xi + jnp.int32(1), jnp.int32(res - 1))
    y0 = jnp.minimum(yi, jnp.int32(res - 1))
    y1 = jnp.minimum(yi + jnp.int32(1), jnp.int32(res - 1))
    x0lo, x0hi = _hprod2(x0, P1_HI, P1_LO, need_hi)
    x1lo, x1hi = _hprod2(x1, P1_HI, P1_LO, need_hi)
    y0lo, y0hi = _hprod2(y0, P2_HI, P2_LO, need_hi)
    y1lo, y1hi = _hprod2(y1, P2_HI, P2_LO, need_hi)
    c00 = _corner_idx(lev, x0lo, x0hi, y0lo, y0hi)
    c10 = _corner_idx(lev, x1lo, x1hi, y0lo, y0hi)
    c01 = _corner_idx(lev, x0lo, x0hi, y1lo, y1hi)
    c11 = _corner_idx(lev, x1lo, x1hi, y1lo, y1hi)
    return (c00, c10, c01, c11), wx, wy


def _bilerp(t00, t10, t01, t11, wx, wy):
    a = t00 + wx * (t10 - t00)
    b = t01 + wx * (t11 - t01)
    return a + wy * (b - a)


def _load_xy(x_buf, pids):
    px = plsc.load_gather(x_buf, [pids * jnp.int32(2)])
    py = plsc.load_gather(x_buf, [pids * jnp.int32(2) + jnp.int32(1)])
    xn = jnp.clip((px + jnp.float32(1.0)) * jnp.float32(0.5),
                  jnp.float32(0.0), jnp.float32(1.0))
    yn = jnp.clip((py + jnp.float32(1.0)) * jnp.float32(0.5),
                  jnp.float32(0.0), jnp.float32(1.0))
    return xn, yn


@functools.cache
def _build():
    mesh = plsc.VectorSubcoreMesh(core_axis_name="c", subcore_axis_name="s",
                                  num_cores=NC, num_subcores=NS)

    scratch = [pltpu.VMEM((RES_ROWS * F,), jnp.float32)]
    scratch += [pltpu.VMEM((2 * CH * 2,), jnp.float32)]
    scratch += [pltpu.VMEM((CH, 2 * NUM_LEVELS), jnp.float32)]
    scratch += [pltpu.VMEM((2 * CH,), jnp.int32) for _ in range(8 * N_STR)]
    scratch += [pltpu.VMEM((2 * CH,), jnp.float32) for _ in range(8 * N_STR)]
    scratch += [pltpu.VMEM((2 * 2 * N_STR * CH,), jnp.float32)]
    scratch += [pltpu.SemaphoreType.DMA]

    @functools.partial(
        pl.kernel,
        out_type=jax.ShapeDtypeStruct((B, 2 * NUM_LEVELS), jnp.float32),
        mesh=mesh,
        scratch_types=scratch,
        compiler_params=pltpu.CompilerParams(
            needs_layout_passes=False, use_tc_tiling_on_sc=False),
    )
    def hash_embed(x_hbm, *rest):
        tabs = rest[:NUM_LEVELS]
        out_hbm = rest[NUM_LEVELS]
        sc = rest[NUM_LEVELS + 1:]
        res_tab = sc[0]
        x_buf = sc[1]
        out_buf = sc[2]
        idx_refs = sc[3:3 + 8 * N_STR]
        row_refs = sc[3 + 8 * N_STR:3 + 16 * N_STR]
        w_ref = sc[3 + 16 * N_STR]
        sem = sc[3 + 16 * N_STR + 1]

        wid = lax.axis_index("s") * NC + lax.axis_index("c")
        wbase = wid * jnp.int32(PPW)

        iota = lax.iota(jnp.int32, L)

        for l in range(N_RES):
            pltpu.sync_copy(
                tabs[l], res_tab.at[pl.ds(F * _RBASE[l], F * TABLE_SIZES[l])])

        def phase_a(ci, po):
            base = wbase + ci * jnp.int32(CH)
            pltpu.sync_copy(x_hbm.at[pl.ds(base * jnp.int32(2), CH * 2)],
                            x_buf.at[pl.ds(po * jnp.int32(CH * 2), CH * 2)])

            def body(v, c):
                pids = iota + v * jnp.int32(L)
                xoff = po * jnp.int32(CH * 2)
                px = plsc.load_gather(x_buf, [xoff + pids * jnp.int32(2)])
                py = plsc.load_gather(
                    x_buf, [xoff + pids * jnp.int32(2) + jnp.int32(1)])
                xn = jnp.clip((px + jnp.float32(1.0)) * jnp.float32(0.5),
                              jnp.float32(0.0), jnp.float32(1.0))
                yn = jnp.clip((py + jnp.float32(1.0)) * jnp.float32(0.5),
                              jnp.float32(0.0), jnp.float32(1.0))
                so = po * jnp.int32(CH) + v * jnp.int32(L)
                wo = po * jnp.int32(2 * N_STR * CH) + v * jnp.int32(L)
                for l in range(N_RES, NUM_LEVELS):
                    k = (l - N_RES) * 8
                    corners, wx, wy = _level_corners(l, xn, yn)
                    for c4 in range(4):
                        w0 = corners[c4] * jnp.int32(F)
                        idx_refs[k + 2 * c4][pl.ds(so, L)] = w0
                        idx_refs[k + 2 * c4 + 1][pl.ds(so, L)] = w0 + jnp.int32(1)
                    w_ref[pl.ds(wo + (l - N_RES) * CH, L)] = wx
                    w_ref[pl.ds(wo + (N_STR + l - N_RES) * CH, L)] = wy
                return c

            lax.fori_loop(np.int32(0), np.int32(NVR), body, np.int32(0))

        def fire(po):
            off = po * jnp.int32(CH)
            for l in range(N_RES, NUM_LEVELS):
                k = (l - N_RES) * 8
                for j in range(8):
                    pltpu.async_copy(
                        tabs[l].at[idx_refs[k + j].at[pl.ds(off, CH)]],
                        row_refs[k + j].at[pl.ds(off, CH)], sem)

        def drain(po):
            off = po * jnp.int32(CH)
            for l in range(N_RES, NUM_LEVELS):
                k = (l - N_RES) * 8
                for j in range(8):
                    pltpu.make_async_copy(
                        tabs[l].at[idx_refs[k + j].at[pl.ds(off, CH)]],
                        row_refs[k + j].at[pl.ds(off, CH)], sem).wait()

        def phase_b(po):
            def body(v, c):
                pids = iota + v * jnp.int32(L)
                xoff = po * jnp.int32(CH * 2)
                px = plsc.load_gather(x_buf, [xoff + pids * jnp.int32(2)])
                py = plsc.load_gather(
                    x_buf, [xoff + pids * jnp.int32(2) + jnp.int32(1)])
                xn = jnp.clip((px + jnp.float32(1.0)) * jnp.float32(0.5),
                              jnp.float32(0.0), jnp.float32(1.0))
                yn = jnp.clip((py + jnp.float32(1.0)) * jnp.float32(0.5),
                              jnp.float32(0.0), jnp.float32(1.0))
                for l in range(N_RES):
                    corners, wx, wy = _level_corners(l, xn, yn)
                    feats = []
                    for f in range(F):
                        off = jnp.int32(F * _RBASE[l] + f)
                        t = [plsc.load_gather(res_tab, [cc * jnp.int32(F) + off])
                             for cc in corners]
                        feats.append(_bilerp(t[0], t[1], t[2], t[3], wx, wy))
                    for f in range(F):
                        plsc.store_scatter(
                            out_buf, [pids, jnp.full((L,), 2 * l + f, jnp.int32)],
                            feats[f])
                return c

            lax.fori_loop(np.int32(0), np.int32(NVR), body, np.int32(0))

        def phase_c(po):
            def body(v, c):
                pids = iota + v * jnp.int32(L)
                so = po * jnp.int32(CH) + v * jnp.int32(L)
                wo = po * jnp.int32(2 * N_STR * CH) + v * jnp.int32(L)
                for l in range(N_RES, NUM_LEVELS):
                    k = (l - N_RES) * 8
                    wx = w_ref[pl.ds(wo + (l - N_RES) * CH, L)]
                    wy = w_ref[pl.ds(wo + (N_STR + l - N_RES) * CH, L)]
                    for f in range(F):
                        t = [row_refs[k + 2 * c4 + f][pl.ds(so, L)]
                             for c4 in range(4)]
                        val = _bilerp(t[0], t[1], t[2], t[3], wx, wy)
                        plsc.store_scatter(
                            out_buf, [pids, jnp.full((L,), 2 * l + f, jnp.int32)],
                            val)
                return c

            lax.fori_loop(np.int32(0), np.int32(NVR), body, np.int32(0))

        phase_a(jnp.int32(0), jnp.int32(0))
        fire(jnp.int32(0))

        def chunk_body(ci, carry):
            par = lax.rem(ci, jnp.int32(2))
            nxt = jnp.int32(1) - par

            @pl.when(ci < jnp.int32(NCHUNK - 1))
            def _():
                phase_a(ci + jnp.int32(1), nxt)
                fire(nxt)

            phase_b(par)
            drain(par)
            phase_c(par)
            base = wbase + ci * jnp.int32(CH)
            pltpu.sync_copy(out_buf, out_hbm.at[pl.ds(base, CH)])
            return carry

        lax.fori_loop(np.int32(0), np.int32(NCHUNK), chunk_body, np.int32(0))

    return hash_embed


def kernel(x, table_0, table_1, table_2, table_3, table_4, table_5, table_6,
           table_7, table_8, table_9, table_10, table_11, table_12,
           table_13, table_14, table_15):
    tabs = [table_0, table_1, table_2, table_3, table_4, table_5, table_6,
            table_7, table_8, table_9, table_10, table_11, table_12,
            table_13, table_14, table_15]
    with jax.enable_x64(False):
        x_flat = x.reshape(B * 2)
        args = [t.reshape(-1) for t in tabs]
        return _build()(x_flat, *args)
